# baseline (speedup 1.0000x reference)
import functools

import jax
import jax.numpy as jnp
from jax import lax
from jax.experimental import pallas as pl
from jax.experimental.pallas import tpu as pltpu
from jax.experimental.pallas import tpu_sc as plsc

B = 8
V = 100000
NC = 2
NS = 16
L = 16
CHUNKS = 4
CW = V // CHUNKS
FULL_STEPS = CW // L
LAST_BASE = CW - L
NSLOT = B * CHUNKS

_mesh = plsc.VectorSubcoreMesh(
    core_axis_name="c", subcore_axis_name="s", num_cores=NC, num_subcores=NS)


def _tile_coords():
    c = lax.axis_index("c")
    s = lax.axis_index("s")
    r = c * (B // NC) + s // CHUNKS
    ch = s % CHUNKS
    return r, ch, r * V + ch * CW


@functools.partial(
    pl.kernel,
    out_type=(
        jax.ShapeDtypeStruct((NSLOT * L,), jnp.float32),
        jax.ShapeDtypeStruct((NSLOT * L,), jnp.int32),
    ),
    mesh=_mesh,
    scratch_types=[
        pltpu.VMEM((CW,), jnp.float32),
        pltpu.VMEM((CW,), jnp.float32),
        pltpu.VMEM((L,), jnp.float32),
        pltpu.VMEM((L,), jnp.int32),
    ],
)
def _scan_kernel(logits_hbm, pen_hbm, cval_out, cidx_out,
                 logits_v, pen_v, tval_v, tidx_v):
    r, ch, woff = _tile_coords()
    pltpu.sync_copy(logits_hbm.at[pl.ds(woff, CW)], logits_v)
    pltpu.sync_copy(pen_hbm.at[pl.ds(woff, CW)], pen_v)

    lanes = lax.iota(jnp.int32, L)

    def step(i, carry):
        bv, bi = carry
        base = jnp.minimum(i * L, LAST_BASE)
        sc = logits_v[pl.ds(base, L)] * pen_v[pl.ds(base, L)]
        idx = lanes + base
        take = sc > bv
        return jnp.where(take, sc, bv), jnp.where(take, idx, bi)

    bv, bi = lax.fori_loop(
        0, FULL_STEPS + 1, step,
        (jnp.full((L,), -jnp.inf, jnp.float32), jnp.zeros((L,), jnp.int32)))

    slot = (r * CHUNKS + ch) * L
    tval_v[...] = bv
    tidx_v[...] = bi + ch * CW
    pltpu.sync_copy(tval_v, cval_out.at[pl.ds(slot, L)])
    pltpu.sync_copy(tidx_v, cidx_out.at[pl.ds(slot, L)])


@functools.partial(
    pl.kernel,
    out_type=(
        jax.ShapeDtypeStruct((B * 8,), jnp.int32),
        jax.ShapeDtypeStruct((B * V,), jnp.float32),
    ),
    mesh=_mesh,
    scratch_types=[
        pltpu.VMEM((CW,), jnp.float32),
        pltpu.VMEM((CHUNKS * L,), jnp.float32),
        pltpu.VMEM((CHUNKS * L,), jnp.int32),
        pltpu.VMEM((L,), jnp.float32),
        pltpu.VMEM((L,), jnp.int32),
    ],
)
def _update_kernel(pen_hbm, cval_hbm, cidx_hbm, scale_hbm, idx_out, pen_out,
                   pen_v, rval_v, ridx_v, scale_v, widx_v):
    r, ch, woff = _tile_coords()
    pltpu.sync_copy(pen_hbm.at[pl.ds(woff, CW)], pen_v)
    pltpu.sync_copy(cval_hbm.at[pl.ds(r * CHUNKS * L, CHUNKS * L)], rval_v)
    pltpu.sync_copy(cidx_hbm.at[pl.ds(r * CHUNKS * L, CHUNKS * L)], ridx_v)
    pltpu.sync_copy(scale_hbm, scale_v)

    lanes = lax.iota(jnp.int32, L)

    bv = rval_v[pl.ds(0, L)]
    bi = ridx_v[pl.ds(0, L)]
    for j in range(1, CHUNKS):
        v = rval_v[pl.ds(j * L, L)]
        vi = ridx_v[pl.ds(j * L, L)]
        take = (v > bv) | ((v == bv) & (vi < bi))
        bv = jnp.where(take, v, bv)
        bi = jnp.where(take, vi, bi)

    best_v = bv[0]
    best_i = bi[0]
    for l in range(1, L):
        v = bv[l]
        i = bi[l]
        take = (v > best_v) | ((v == best_v) & (i < best_i))
        best_v = jnp.where(take, v, best_v)
        best_i = jnp.where(take, i, best_i)
    g = best_i

    owner = g // CW

    @pl.when(ch == owner)
    def _update():
        loc = g - owner * CW
        vbase = jnp.minimum(loc & ~(L - 1), CW - L)
        off = loc - vbase
        cur = pen_v[pl.ds(vbase, L)]
        pen_v[pl.ds(vbase, L)] = jnp.where(lanes == off, cur * scale_v[...], cur)

    pltpu.sync_copy(pen_v, pen_out.at[pl.ds(woff, CW)])

    @pl.when(ch == 0)
    def _write_idx():
        widx_v[...] = jnp.full((L,), g, jnp.int32)
        pltpu.sync_copy(widx_v.at[pl.ds(0, 8)], idx_out.at[pl.ds(r * 8, 8)])


def kernel(logits, repeat_penality, penality_value, batch_size):
    del batch_size
    logits_flat = logits.reshape(B * V)
    pen_flat = repeat_penality.reshape(B * V)
    scale = jnp.full((L,), penality_value, jnp.float32)
    cvals, cidxs = _scan_kernel(logits_flat, pen_flat)
    idx_slots, new_pen = _update_kernel(pen_flat, cvals, cidxs, scale)
    return idx_slots[::8].reshape(B, 1), new_pen.reshape(B, V)


# speedup vs baseline: 1.1301x; 1.1301x over previous
import functools

import jax
import jax.numpy as jnp
from jax import lax
from jax.experimental import pallas as pl
from jax.experimental.pallas import tpu as pltpu
from jax.experimental.pallas import tpu_sc as plsc

B = 8
V = 100000
NC = 2
NS = 16
L = 16
CHUNKS = 4
CW = V // CHUNKS
FULL_STEPS = CW // L
LAST_BASE = CW - L
NSLOT = B * CHUNKS

_mesh = plsc.VectorSubcoreMesh(
    core_axis_name="c", subcore_axis_name="s", num_cores=NC, num_subcores=NS)


def _tile_coords():
    c = lax.axis_index("c")
    s = lax.axis_index("s")
    r = c * (B // NC) + s // CHUNKS
    ch = s % CHUNKS
    return r, ch, ch * CW


@functools.partial(
    pl.kernel,
    out_type=(
        jax.ShapeDtypeStruct((NSLOT * L,), jnp.float32),
        jax.ShapeDtypeStruct((NSLOT * L,), jnp.int32),
    ),
    mesh=_mesh,
    scratch_types=[
        pltpu.VMEM((CW,), jnp.float32),
        pltpu.VMEM((CW,), jnp.float32),
        pltpu.VMEM((L,), jnp.float32),
        pltpu.VMEM((L,), jnp.int32),
        pltpu.SemaphoreType.DMA,
        pltpu.SemaphoreType.DMA,
    ],
)
def _scan_kernel(logits_hbm, pen_hbm, cval_out, cidx_out,
                 logits_v, pen_v, tval_v, tidx_v, sem1, sem2):
    r, ch, colbase = _tile_coords()
    woff = r * V + colbase
    d1 = pltpu.async_copy(logits_hbm.at[pl.ds(woff, CW)], logits_v, sem1)
    d2 = pltpu.async_copy(pen_hbm.at[pl.ds(woff, CW)], pen_v, sem2)
    d1.wait()
    d2.wait()

    lanes = lax.iota(jnp.int32, L)
    ninf = jnp.full((L,), -jnp.inf, jnp.float32)

    def step(i, carry):
        bv, bb = carry
        base = i * L
        sc = logits_v[pl.ds(base, L)] * pen_v[pl.ds(base, L)]
        take = sc > bv
        return jnp.where(take, sc, bv), jnp.where(take, base, bb)

    bv, bb = lax.fori_loop(
        0, FULL_STEPS, step, (ninf, jnp.zeros((L,), jnp.int32)), unroll=8)
    sc = logits_v[pl.ds(LAST_BASE, L)] * pen_v[pl.ds(LAST_BASE, L)]
    take = sc > bv
    bv = jnp.where(take, sc, bv)
    bb = jnp.where(take, jnp.int32(LAST_BASE), bb)

    slot = (r * CHUNKS + ch) * L
    tval_v[...] = bv
    tidx_v[...] = bb + lanes + ch * CW
    pltpu.sync_copy(tval_v, cval_out.at[pl.ds(slot, L)])
    pltpu.sync_copy(tidx_v, cidx_out.at[pl.ds(slot, L)])


@functools.partial(
    pl.kernel,
    out_type=(
        jax.ShapeDtypeStruct((B * 8,), jnp.int32),
        jax.ShapeDtypeStruct((B * V,), jnp.float32),
    ),
    mesh=_mesh,
    scratch_types=[
        pltpu.VMEM((CW,), jnp.float32),
        pltpu.VMEM((CHUNKS * L,), jnp.float32),
        pltpu.VMEM((CHUNKS * L,), jnp.int32),
        pltpu.VMEM((L,), jnp.float32),
        pltpu.VMEM((L,), jnp.int32),
        pltpu.SemaphoreType.DMA,
    ],
)
def _update_kernel(pen_hbm, cval_hbm, cidx_hbm, scale_hbm, idx_out, pen_out,
                   pen_v, rval_v, ridx_v, scale_v, widx_v, semc):
    r, ch, colbase = _tile_coords()
    woff = r * V + colbase
    big = pltpu.async_copy(pen_hbm.at[pl.ds(woff, CW)], pen_v, semc)
    pltpu.sync_copy(cval_hbm.at[pl.ds(r * CHUNKS * L, CHUNKS * L)], rval_v)
    pltpu.sync_copy(cidx_hbm.at[pl.ds(r * CHUNKS * L, CHUNKS * L)], ridx_v)
    pltpu.sync_copy(scale_hbm, scale_v)

    lanes = lax.iota(jnp.int32, L)

    bv = rval_v[pl.ds(0, L)]
    bi = ridx_v[pl.ds(0, L)]
    for j in range(1, CHUNKS):
        v = rval_v[pl.ds(j * L, L)]
        vi = ridx_v[pl.ds(j * L, L)]
        take = (v > bv) | ((v == bv) & (vi < bi))
        bv = jnp.where(take, v, bv)
        bi = jnp.where(take, vi, bi)

    best_v = bv[0]
    best_i = bi[0]
    for l in range(1, L):
        v = bv[l]
        i = bi[l]
        take = (v > best_v) | ((v == best_v) & (i < best_i))
        best_v = jnp.where(take, v, best_v)
        best_i = jnp.where(take, i, best_i)
    g = best_i

    owner = g // CW
    big.wait()

    @pl.when(ch == owner)
    def _fix():
        loc = g - owner * CW
        vbase = jnp.minimum(loc & ~(L - 1), CW - L)
        off = loc - vbase
        cur = pen_v[pl.ds(vbase, L)]
        pen_v[pl.ds(vbase, L)] = jnp.where(lanes == off, cur * scale_v[...], cur)

    pltpu.sync_copy(pen_v, pen_out.at[pl.ds(woff, CW)])

    @pl.when(ch == 0)
    def _write_idx():
        widx_v[...] = jnp.full((L,), g, jnp.int32)
        pltpu.sync_copy(widx_v.at[pl.ds(0, 8)], idx_out.at[pl.ds(r * 8, 8)])


def kernel(logits, repeat_penality, penality_value, batch_size):
    del batch_size
    logits_flat = logits.reshape(B * V)
    pen_flat = repeat_penality.reshape(B * V)
    scale = jnp.full((L,), penality_value, jnp.float32)
    cvals, cidxs = _scan_kernel(logits_flat, pen_flat)
    idx_slots, new_pen = _update_kernel(pen_flat, cvals, cidxs, scale)
    return idx_slots[::8].reshape(B, 1), new_pen.reshape(B, V)


# speedup vs baseline: 1.2945x; 1.1455x over previous
import functools

import jax
import jax.numpy as jnp
from jax import lax
from jax.experimental import pallas as pl
from jax.experimental.pallas import tpu as pltpu
from jax.experimental.pallas import tpu_sc as plsc

B = 8
V = 100000
NC = 2
NS = 16
NW = NC * NS
L = 16
TILE = 128
FULL_TILES = V // TILE
TAIL = V - FULL_TILES * TILE
TAIL_COL = FULL_TILES * TILE
VPAD = (FULL_TILES + 1) * TILE
WT = 25
SW = WT * TILE
VPS = SW // L
LAST_TB = FULL_TILES - WT

_mesh = plsc.VectorSubcoreMesh(
    core_axis_name="c", subcore_axis_name="s", num_cores=NC, num_subcores=NS)


def _worker_id():
    return lax.axis_index("c") * NS + lax.axis_index("s")


def _stripe_base(w):
    return jnp.minimum(w * WT, LAST_TB) * TILE


def _merge_rows(rval_v, ridx_v):
    winners = []
    for r in range(B):
        bv = rval_v[pl.ds(r * NW * L, L)]
        bi = ridx_v[pl.ds(r * NW * L, L)]
        for wsl in range(1, NW):
            v = rval_v[pl.ds((r * NW + wsl) * L, L)]
            vi = ridx_v[pl.ds((r * NW + wsl) * L, L)]
            take = (v > bv) | ((v == bv) & (vi < bi))
            bv = jnp.where(take, v, bv)
            bi = jnp.where(take, vi, bi)
        best_v = bv[0]
        best_i = bi[0]
        for l in range(1, L):
            v = bv[l]
            i = bi[l]
            take = (v > best_v) | ((v == best_v) & (i < best_i))
            best_v = jnp.where(take, v, best_v)
            best_i = jnp.where(take, i, best_i)
        winners.append((best_v, best_i))
    return winners


@functools.partial(
    pl.kernel,
    out_type=(
        jax.ShapeDtypeStruct((B * NW * L,), jnp.float32),
        jax.ShapeDtypeStruct((B * NW * L,), jnp.int32),
    ),
    mesh=_mesh,
    scratch_types=[
        pltpu.VMEM((B, SW), jnp.float32),
        pltpu.VMEM((B, SW), jnp.float32),
        pltpu.VMEM((B * TAIL,), jnp.float32),
        pltpu.VMEM((B * TAIL,), jnp.float32),
        pltpu.VMEM((L,), jnp.float32),
        pltpu.VMEM((L,), jnp.int32),
        pltpu.SemaphoreType.DMA,
        pltpu.SemaphoreType.DMA,
        pltpu.SemaphoreType.DMA,
        pltpu.SemaphoreType.DMA,
    ],
)
def _scan_kernel(logits_hbm, pen_hbm, ltail_hbm, ptail_hbm,
                 cval_out, cidx_out,
                 logits_v, pen_v, tlog_v, tpen_v, tval_v, tidx_v,
                 sem1, sem2, sem3, sem4):
    w = _worker_id()
    cb = pl.multiple_of(_stripe_base(w), TILE)
    d1 = pltpu.async_copy(logits_hbm.at[:, pl.ds(cb, SW)], logits_v, sem1)
    d2 = pltpu.async_copy(pen_hbm.at[:, pl.ds(cb, SW)], pen_v, sem2)
    d3 = pltpu.async_copy(ltail_hbm, tlog_v, sem3)
    d4 = pltpu.async_copy(ptail_hbm, tpen_v, sem4)
    d1.wait()
    d2.wait()
    d3.wait()
    d4.wait()

    lanes = lax.iota(jnp.int32, L)
    ninf = jnp.full((L,), -jnp.inf, jnp.float32)
    zero = jnp.zeros((L,), jnp.int32)

    @pl.when(w < NW - 1)
    def _poison_tail():
        one = jnp.full((L,), 1.0, jnp.float32)
        for i in range(B * TAIL // L):
            tlog_v[pl.ds(i * L, L)] = ninf
            tpen_v[pl.ds(i * L, L)] = one

    def step(k, carry):
        base = pl.multiple_of(k * L, L)
        out = []
        for r in range(B):
            bv, bb = carry[r]
            sc = logits_v[r, pl.ds(base, L)] * pen_v[r, pl.ds(base, L)]
            take = sc > bv
            out.append((jnp.where(take, sc, bv), jnp.where(take, base, bb)))
        return tuple(out)

    carry = lax.fori_loop(0, VPS, step, tuple((ninf, zero) for _ in range(B)))

    for r in range(B):
        bv, bb = carry[r]
        tv0 = tlog_v[pl.ds(r * TAIL, L)] * tpen_v[pl.ds(r * TAIL, L)]
        tv1 = tlog_v[pl.ds(r * TAIL + L, L)] * tpen_v[pl.ds(r * TAIL + L, L)]
        m0 = tv0 > bv
        bv = jnp.where(m0, tv0, bv)
        bb = jnp.where(m0, jnp.int32(TAIL_COL) - cb, bb)
        m1 = tv1 > bv
        bv = jnp.where(m1, tv1, bv)
        bb = jnp.where(m1, jnp.int32(TAIL_COL + L) - cb, bb)
        tval_v[...] = bv
        tidx_v[...] = bb + lanes + cb
        slot = (r * NW + w) * L
        pltpu.sync_copy(tval_v, cval_out.at[pl.ds(slot, L)])
        pltpu.sync_copy(tidx_v, cidx_out.at[pl.ds(slot, L)])


@functools.partial(
    pl.kernel,
    out_type=(
        jax.ShapeDtypeStruct((B,), jnp.int32),
        jax.ShapeDtypeStruct((B, VPAD), jnp.float32),
    ),
    mesh=_mesh,
    scratch_types=[
        pltpu.VMEM((B, SW), jnp.float32),
        pltpu.VMEM((B * TAIL,), jnp.float32),
        pltpu.VMEM((B, TILE), jnp.float32),
        pltpu.VMEM((B * NW * L,), jnp.float32),
        pltpu.VMEM((B * NW * L,), jnp.int32),
        pltpu.VMEM((L,), jnp.float32),
        pltpu.VMEM((L,), jnp.int32),
        pltpu.SemaphoreType.DMA,
        pltpu.SemaphoreType.DMA,
    ],
)
def _update_kernel(pen_hbm, ptail_hbm, cval_hbm, cidx_hbm, scale_hbm,
                   idx_out, pen_out,
                   pen_v, tpen_v, tout_v, rval_v, ridx_v, scale_v, widx_v,
                   semc, semt):
    w = _worker_id()
    cb = pl.multiple_of(_stripe_base(w), TILE)
    is_last = w == NW - 1
    big = pltpu.async_copy(pen_hbm.at[:, pl.ds(cb, SW)], pen_v, semc)
    tail = pltpu.async_copy(ptail_hbm, tpen_v, semt)
    pltpu.sync_copy(cval_hbm, rval_v)
    pltpu.sync_copy(cidx_hbm, ridx_v)
    pltpu.sync_copy(scale_hbm, scale_v)

    lanes = lax.iota(jnp.int32, L)
    winners = _merge_rows(rval_v, ridx_v)
    big.wait()
    tail.wait()

    for r in range(B):
        _, g = winners[r]
        loc = g - cb
        in_stripe = (loc >= 0) & (loc < SW)

        @pl.when(in_stripe)
        def _fix(r=r, loc=loc):
            vbase = pl.multiple_of(loc & ~(L - 1), L)
            off = loc - vbase
            cur = pen_v[r, pl.ds(vbase, L)]
            pen_v[r, pl.ds(vbase, L)] = jnp.where(
                lanes == off, cur * scale_v[...], cur)

        in_tail = is_last & (g >= TAIL_COL)

        @pl.when(in_tail)
        def _fix_tail(r=r, g=g):
            tloc = g - TAIL_COL
            tb = pl.multiple_of((r * TAIL + tloc) & ~(L - 1), L)
            toff = (r * TAIL + tloc) - tb
            cur = tpen_v[pl.ds(tb, L)]
            tpen_v[pl.ds(tb, L)] = jnp.where(
                lanes == toff, cur * scale_v[...], cur)

    pltpu.sync_copy(pen_v, pen_out.at[:, pl.ds(cb, SW)])

    @pl.when(is_last)
    def _tail_out():
        for r in range(B):
            tout_v[r, pl.ds(0, L)] = tpen_v[pl.ds(r * TAIL, L)]
            tout_v[r, pl.ds(L, L)] = tpen_v[pl.ds(r * TAIL + L, L)]
        pltpu.sync_copy(tout_v, pen_out.at[:, pl.ds(TAIL_COL, TILE)])

    @pl.when(w == 0)
    def _write_idx():
        vi = jnp.zeros((L,), jnp.int32)
        for r in range(B):
            _, g = winners[r]
            vi = jnp.where(lanes == r, g, vi)
        widx_v[...] = vi
        pltpu.sync_copy(widx_v.at[pl.ds(0, 8)], idx_out)


def kernel(logits, repeat_penality, penality_value, batch_size):
    del batch_size
    ltail = lax.slice(logits, (0, TAIL_COL), (B, V)).reshape(B * TAIL)
    ptail = lax.slice(repeat_penality, (0, TAIL_COL), (B, V)).reshape(B * TAIL)
    scale = jnp.full((L,), penality_value, jnp.float32)
    cvals, cidxs = _scan_kernel(logits, repeat_penality, ltail, ptail)
    idx8, pen_pad = _update_kernel(repeat_penality, ptail, cvals, cidxs, scale)
    return idx8.reshape(B, 1), lax.slice(pen_pad, (0, 0), (B, V))


# speedup vs baseline: 1.3095x; 1.0116x over previous
import functools

import jax
import jax.numpy as jnp
from jax import lax
from jax.experimental import pallas as pl
from jax.experimental.pallas import tpu as pltpu
from jax.experimental.pallas import tpu_sc as plsc

B = 8
V = 100000
NC = 2
NS = 16
NW = NC * NS
L = 16
TILE = 128
FULL_TILES = V // TILE
TAIL = V - FULL_TILES * TILE
TAIL_COL = FULL_TILES * TILE
VPAD = (FULL_TILES + 1) * TILE
WT = 25
SW = WT * TILE
VPS = SW // L
LAST_TB = FULL_TILES - WT

_mesh = plsc.VectorSubcoreMesh(
    core_axis_name="c", subcore_axis_name="s", num_cores=NC, num_subcores=NS)


def _worker_id():
    return lax.axis_index("c") * NS + lax.axis_index("s")


def _stripe_base(w):
    return jnp.minimum(w * WT, LAST_TB) * TILE


def _merge_rows(rval_v, ridx_v):
    winners = []
    for r in range(B):
        bv = rval_v[pl.ds(r * NW * L, L)]
        bi = ridx_v[pl.ds(r * NW * L, L)]
        for wsl in range(1, NW):
            v = rval_v[pl.ds((r * NW + wsl) * L, L)]
            vi = ridx_v[pl.ds((r * NW + wsl) * L, L)]
            take = (v > bv) | ((v == bv) & (vi < bi))
            bv = jnp.where(take, v, bv)
            bi = jnp.where(take, vi, bi)
        best_v = bv[0]
        best_i = bi[0]
        for l in range(1, L):
            v = bv[l]
            i = bi[l]
            take = (v > best_v) | ((v == best_v) & (i < best_i))
            best_v = jnp.where(take, v, best_v)
            best_i = jnp.where(take, i, best_i)
        winners.append((best_v, best_i))
    return winners


@functools.partial(
    pl.kernel,
    out_type=(
        jax.ShapeDtypeStruct((B * NW * L,), jnp.float32),
        jax.ShapeDtypeStruct((B * NW * L,), jnp.int32),
    ),
    mesh=_mesh,
    scratch_types=[
        pltpu.VMEM((B, SW), jnp.float32),
        pltpu.VMEM((B, SW), jnp.float32),
        pltpu.VMEM((B * TAIL,), jnp.float32),
        pltpu.VMEM((B * TAIL,), jnp.float32),
        pltpu.VMEM((L,), jnp.float32),
        pltpu.VMEM((L,), jnp.int32),
        pltpu.SemaphoreType.DMA,
        pltpu.SemaphoreType.DMA,
        pltpu.SemaphoreType.DMA,
        pltpu.SemaphoreType.DMA,
    ],
)
def _scan_kernel(logits_hbm, pen_hbm, ltail_hbm, ptail_hbm,
                 cval_out, cidx_out,
                 logits_v, pen_v, tlog_v, tpen_v, tval_v, tidx_v,
                 sem1, sem2, sem3, sem4):
    w = _worker_id()
    cb = pl.multiple_of(_stripe_base(w), TILE)
    d1 = pltpu.async_copy(logits_hbm.at[:, pl.ds(cb, SW)], logits_v, sem1)
    d2 = pltpu.async_copy(pen_hbm.at[:, pl.ds(cb, SW)], pen_v, sem2)
    d3 = pltpu.async_copy(ltail_hbm, tlog_v, sem3)
    d4 = pltpu.async_copy(ptail_hbm, tpen_v, sem4)
    d1.wait()
    d2.wait()
    d3.wait()
    d4.wait()

    lanes = lax.iota(jnp.int32, L)
    ninf = jnp.full((L,), -jnp.inf, jnp.float32)
    zero = jnp.zeros((L,), jnp.int32)

    @pl.when(w < NW - 1)
    def _poison_tail():
        one = jnp.full((L,), 1.0, jnp.float32)
        for i in range(B * TAIL // L):
            tlog_v[pl.ds(i * L, L)] = ninf
            tpen_v[pl.ds(i * L, L)] = one

    def step(k, carry):
        base = pl.multiple_of(k * L, L)
        out = []
        for r in range(B):
            bv, bb = carry[r]
            sc = logits_v[r, pl.ds(base, L)] * pen_v[r, pl.ds(base, L)]
            take = sc > bv
            out.append((jnp.where(take, sc, bv), jnp.where(take, base, bb)))
        return tuple(out)

    carry = lax.fori_loop(0, VPS, step, tuple((ninf, zero) for _ in range(B)),
                          unroll=4)

    for r in range(B):
        bv, bb = carry[r]
        tv0 = tlog_v[pl.ds(r * TAIL, L)] * tpen_v[pl.ds(r * TAIL, L)]
        tv1 = tlog_v[pl.ds(r * TAIL + L, L)] * tpen_v[pl.ds(r * TAIL + L, L)]
        m0 = tv0 > bv
        bv = jnp.where(m0, tv0, bv)
        bb = jnp.where(m0, jnp.int32(TAIL_COL) - cb, bb)
        m1 = tv1 > bv
        bv = jnp.where(m1, tv1, bv)
        bb = jnp.where(m1, jnp.int32(TAIL_COL + L) - cb, bb)
        tval_v[...] = bv
        tidx_v[...] = bb + lanes + cb
        slot = (r * NW + w) * L
        pltpu.sync_copy(tval_v, cval_out.at[pl.ds(slot, L)])
        pltpu.sync_copy(tidx_v, cidx_out.at[pl.ds(slot, L)])


@functools.partial(
    pl.kernel,
    out_type=(
        jax.ShapeDtypeStruct((B,), jnp.int32),
        jax.ShapeDtypeStruct((B, VPAD), jnp.float32),
    ),
    mesh=_mesh,
    scratch_types=[
        pltpu.VMEM((B, SW), jnp.float32),
        pltpu.VMEM((B * TAIL,), jnp.float32),
        pltpu.VMEM((B, TILE), jnp.float32),
        pltpu.VMEM((B * NW * L,), jnp.float32),
        pltpu.VMEM((B * NW * L,), jnp.int32),
        pltpu.VMEM((L,), jnp.float32),
        pltpu.VMEM((L,), jnp.int32),
        pltpu.SemaphoreType.DMA,
        pltpu.SemaphoreType.DMA,
    ],
)
def _update_kernel(pen_hbm, ptail_hbm, cval_hbm, cidx_hbm, scale_hbm,
                   idx_out, pen_out,
                   pen_v, tpen_v, tout_v, rval_v, ridx_v, scale_v, widx_v,
                   semc, semt):
    w = _worker_id()
    cb = pl.multiple_of(_stripe_base(w), TILE)
    is_last = w == NW - 1
    big = pltpu.async_copy(pen_hbm.at[:, pl.ds(cb, SW)], pen_v, semc)
    tail = pltpu.async_copy(ptail_hbm, tpen_v, semt)
    pltpu.sync_copy(cval_hbm, rval_v)
    pltpu.sync_copy(cidx_hbm, ridx_v)
    pltpu.sync_copy(scale_hbm, scale_v)

    lanes = lax.iota(jnp.int32, L)
    winners = _merge_rows(rval_v, ridx_v)
    big.wait()
    tail.wait()

    for r in range(B):
        _, g = winners[r]
        loc = g - cb
        in_stripe = (loc >= 0) & (loc < SW)

        @pl.when(in_stripe)
        def _fix(r=r, loc=loc):
            vbase = pl.multiple_of(loc & ~(L - 1), L)
            off = loc - vbase
            cur = pen_v[r, pl.ds(vbase, L)]
            pen_v[r, pl.ds(vbase, L)] = jnp.where(
                lanes == off, cur * scale_v[...], cur)

        in_tail = is_last & (g >= TAIL_COL)

        @pl.when(in_tail)
        def _fix_tail(r=r, g=g):
            tloc = g - TAIL_COL
            tb = pl.multiple_of((r * TAIL + tloc) & ~(L - 1), L)
            toff = (r * TAIL + tloc) - tb
            cur = tpen_v[pl.ds(tb, L)]
            tpen_v[pl.ds(tb, L)] = jnp.where(
                lanes == toff, cur * scale_v[...], cur)

    pltpu.sync_copy(pen_v, pen_out.at[:, pl.ds(cb, SW)])

    @pl.when(is_last)
    def _tail_out():
        for r in range(B):
            tout_v[r, pl.ds(0, L)] = tpen_v[pl.ds(r * TAIL, L)]
            tout_v[r, pl.ds(L, L)] = tpen_v[pl.ds(r * TAIL + L, L)]
        pltpu.sync_copy(tout_v, pen_out.at[:, pl.ds(TAIL_COL, TILE)])

    @pl.when(w == 0)
    def _write_idx():
        vi = jnp.zeros((L,), jnp.int32)
        for r in range(B):
            _, g = winners[r]
            vi = jnp.where(lanes == r, g, vi)
        widx_v[...] = vi
        pltpu.sync_copy(widx_v.at[pl.ds(0, 8)], idx_out)


def kernel(logits, repeat_penality, penality_value, batch_size):
    del batch_size
    ltail = lax.slice(logits, (0, TAIL_COL), (B, V)).reshape(B * TAIL)
    ptail = lax.slice(repeat_penality, (0, TAIL_COL), (B, V)).reshape(B * TAIL)
    scale = jnp.full((L,), penality_value, jnp.float32)
    cvals, cidxs = _scan_kernel(logits, repeat_penality, ltail, ptail)
    idx8, pen_pad = _update_kernel(repeat_penality, ptail, cvals, cidxs, scale)
    return idx8.reshape(B, 1), lax.slice(pen_pad, (0, 0), (B, V))
